# baseline (speedup 1.0000x reference)
import functools

import jax
import jax.numpy as jnp
from jax.experimental import pallas as pl
from jax.experimental.pallas import tpu as pltpu


def _gcn_body(m_ref, x_ref, w_ref, o_ref):
    acc = jnp.dot(m_ref[...], x_ref[...],
                  preferred_element_type=jnp.float32)
    o_ref[...] = acc * w_ref[...]


@functools.partial(jax.jit, static_argnames=("block_m",))
def _gcn_layer(inlayer, M, W, block_m=240):
    n, d = inlayer.shape
    grid = (pl.cdiv(M.shape[0], block_m),)
    return pl.pallas_call(
        _gcn_body,
        grid=grid,
        in_specs=[
            pl.BlockSpec((block_m, n), lambda i: (i, 0)),
            pl.BlockSpec((n, d), lambda i: (0, 0)),
            pl.BlockSpec((1, d), lambda i: (0, 0)),
        ],
        out_specs=pl.BlockSpec((block_m, d), lambda i: (i, 0)),
        out_shape=jax.ShapeDtypeStruct((M.shape[0], d), jnp.float32),
        compiler_params=pltpu.CompilerParams(
            dimension_semantics=("parallel",),
        ),
    )(M, inlayer, W)


def kernel(inlayer, M, W):
    return _gcn_layer(inlayer, M, W)


# speedup vs baseline: 1.0072x; 1.0072x over previous
import functools

import jax
import jax.numpy as jnp
from jax.experimental import pallas as pl
from jax.experimental.pallas import tpu as pltpu


def _gcn_body(m_ref, x_ref, w_ref, o_ref):
    acc = jnp.dot(m_ref[...], x_ref[...],
                  preferred_element_type=jnp.float32)
    o_ref[...] = acc * w_ref[...]


@functools.partial(jax.jit, static_argnames=("block_m",))
def _gcn_layer(inlayer, M, W, block_m=240):
    n, d = inlayer.shape
    grid = (pl.cdiv(M.shape[0], block_m),)
    return pl.pallas_call(
        _gcn_body,
        grid=grid,
        in_specs=[
            pl.BlockSpec((block_m, n), lambda i: (i, 0)),
            pl.BlockSpec((n, d), lambda i: (0, 0)),
            pl.BlockSpec((1, d), lambda i: (0, 0)),
        ],
        out_specs=pl.BlockSpec((block_m, d), lambda i: (i, 0)),
        out_shape=jax.ShapeDtypeStruct((M.shape[0], d), jnp.float32),
        compiler_params=pltpu.CompilerParams(
            dimension_semantics=("arbitrary",),
        ),
    )(M, inlayer, W)


def kernel(inlayer, M, W):
    return _gcn_layer(inlayer, M, W)
